# baseline (speedup 1.0000x reference)
import dataclasses

import jax
import jax.numpy as jnp
from jax import lax
from jax.experimental import pallas as pl
from jax.experimental.pallas import tpu as pltpu
from jax.experimental.pallas import tpu_sc as plsc

_B = 16384
_F = 26
_K = 16
_FEAT = 100000
_NC = 2
_NS = 16
_NW = _NC * _NS
_EPW = _B // _NW
_C = 64
_NCH = _EPW // _C
_IPC = _C * _F
_IROWS = _IPC // 128
_G = _C // 16


def _fm_body(idx_hbm, val_hbm, offs_hbm, w_hbm, v_hbm, out_hbm,
             idxv, idxw, offsv, valv, vrows, wrot, rbuf, outbuf, semv, semw):
    wid = lax.axis_index("s") * _NC + lax.axis_index("c")
    pltpu.sync_copy(offs_hbm, offsv)

    iota = lax.iota(jnp.int32, 16)
    iota_f = iota * _F
    iota_k = iota * 17
    zeros_i = jnp.zeros((16,), jnp.int32)
    zero = jnp.zeros((16,), jnp.float32)

    @pl.loop(0, _NCH)
    def _chunk(ch):
        ebase = wid * _EPW + ch * _C
        pltpu.sync_copy(idx_hbm.at[pl.ds(ebase * _F, _IPC)], idxv)
        pltpu.sync_copy(val_hbm.at[pl.ds(ebase * _F, _IPC)], valv)

        @pl.loop(0, _IPC // 16)
        def _off(j):
            sl = pl.ds(j * 16, 16)
            full = idxv[sl] + offsv[sl]
            idxv[sl] = full
            idxw[sl] = lax.shift_right_logical(full, 4)

        cps = []
        for j in range(_IROWS):
            isl = pl.ds(j * 128, 128)
            dst = pl.ds(j * 128, 128)
            cps.append(pltpu.async_copy(v_hbm.at[idxv.at[isl]], vrows.at[dst], semv))
            cps.append(pltpu.async_copy(w_hbm.at[idxw.at[isl]], wrot.at[dst], semw))
        for cp in cps:
            cp.wait()

        @pl.loop(0, _G)
        def _group(g):
            lg = g * 16

            @pl.loop(0, 16)
            def _elem(e16):
                vb = (lg + e16) * _F
                acc = zero
                acc2 = zero
                for f in range(_F):
                    row = vrows[vb + f, :]
                    v = plsc.load_gather(valv, [zeros_i + (vb + f)])
                    rv = row * v
                    acc = acc + rv
                    acc2 = acc2 + rv * rv
                rbuf[pl.ds(e16 * 17, 16)] = acc * acc - acc2

            facc = zero
            gflat = lg * _F
            for f in range(_F):
                pos = iota_f + (gflat + f)
                lane = plsc.load_gather(idxv, [pos]) & 15
                wv = plsc.load_gather(wrot, [pos, lane])
                vv = plsc.load_gather(valv, [pos])
                facc = facc + wv * vv

            s = zero
            for k in range(16):
                s = s + plsc.load_gather(rbuf, [iota_k + k])

            outbuf[pl.ds(ch * _C + lg, 16)] = facc + 0.5 * s

    pltpu.sync_copy(outbuf, out_hbm.at[pl.ds(wid * _EPW, _EPW)])


def kernel(inputs_index, inputs_value, w0, w, V):
    idxflat = inputs_index.astype(jnp.int32).reshape(_B * _F)
    valflat = inputs_value.reshape(_B * _F)
    offs = jnp.tile(jnp.arange(_F, dtype=jnp.int32) * _FEAT, _C)

    mesh = plsc.VectorSubcoreMesh(core_axis_name="c", subcore_axis_name="s",
                                  num_cores=_NC, num_subcores=_NS)
    cp = pltpu.CompilerParams()
    if "needs_layout_passes" in pltpu.CompilerParams.__dataclass_fields__:
        cp = dataclasses.replace(cp, needs_layout_passes=False)
    cp = dataclasses.replace(cp, use_tc_tiling_on_sc=False)
    fm = pl.kernel(
        _fm_body,
        out_type=jax.ShapeDtypeStruct((_B,), jnp.float32),
        mesh=mesh,
        scratch_types=[
            pltpu.VMEM((_IPC,), jnp.int32),
            pltpu.VMEM((_IPC,), jnp.int32),
            pltpu.VMEM((_IPC,), jnp.int32),
            pltpu.VMEM((_IPC,), jnp.float32),
            pltpu.VMEM((_IPC, _K), jnp.float32),
            pltpu.VMEM((_IPC, 16), jnp.float32),
            pltpu.VMEM((16 * 17,), jnp.float32),
            pltpu.VMEM((_EPW,), jnp.float32),
            pltpu.SemaphoreType.DMA,
            pltpu.SemaphoreType.DMA,
        ],
        compiler_params=cp,
    )
    out = fm(idxflat, valflat, offs, w.reshape(-1, 16), V)
    return out.reshape(_B, 1) + w0
